# Initial kernel scaffold; baseline (speedup 1.0000x reference)
import jax
import jax.numpy as jnp
from jax.experimental import pallas as pl


def kernel(x, edge_index, batch, pegs_left, move_count, Wl1, Wr1, att1, b1, Wl2, Wr2, att2, b2, vW1, vb1, vW2, vb2, pW1, pb1, pW2, pb2):
    raise NotImplementedError("write your pallas kernel here")


# speedup vs baseline: 47.4165x; 47.4165x over previous
import functools

import jax
import jax.numpy as jnp
from jax import lax
from jax.experimental import pallas as pl
from jax.experimental.pallas import tpu as pltpu
from jax.experimental.pallas import tpu_sc as plsc

N_TC = 10240
E_TOT = 320000
NSC = 2
NTILE = 16
L = 16
D1 = 256
DH = 128
D2 = 64
ROW1 = DH + 16
ROW2 = D2 + 16
BN = 1024
NB = N_TC // BN
ECH = 4000


def _mesh():
    return plsc.VectorSubcoreMesh(core_axis_name="c", subcore_axis_name="s")


def _tc1_body(x_ref, wl_ref, wr_ref, xl_ref, xr_ref):
    xb = x_ref[...]
    xl_ref[...] = jnp.dot(xb, wl_ref[...], preferred_element_type=jnp.float32)
    xr_ref[...] = jnp.dot(xb, wr_ref[...], preferred_element_type=jnp.float32)


def _tc1(x_pad, Wl1, Wr1):
    return pl.pallas_call(
        _tc1_body,
        grid=(2, NB),
        in_specs=[
            pl.BlockSpec((BN, 128), lambda c, i: (i, 0)),
            pl.BlockSpec((128, DH), lambda c, i: (0, c)),
            pl.BlockSpec((128, DH), lambda c, i: (0, c)),
        ],
        out_specs=[
            pl.BlockSpec((BN, DH), lambda c, i: (c * NB + i, 0)),
            pl.BlockSpec((BN, DH), lambda c, i: (c * NB + i, 0)),
        ],
        out_shape=[
            jax.ShapeDtypeStruct((2 * N_TC, DH), jnp.float32),
            jax.ShapeDtypeStruct((2 * N_TC, DH), jnp.float32),
        ],
    )(x_pad, Wl1, Wr1)


def _sc1_body(src_hbm, dst_hbm, xl_hbm, xr_hbm, att_hbm, zro_hbm, out_hbm,
              srcbuf, dstbuf, attbuf, gsrc0, gsrc1, gdst0, gdst1, sidx0, sidx1,
              xlbuf0, xlbuf1, xrbuf0, xrbuf1, msgbuf0, msgbuf1, acc,
              sl0, sl1, sr0, sr1, ss0, ss1):
    c = lax.axis_index("c")
    s = lax.axis_index("s")
    ept = E_TOT // NTILE
    base = s * ept
    gsrc = (gsrc0, gsrc1)
    gdst = (gdst0, gdst1)
    xlb = (xlbuf0, xlbuf1)
    xrb = (xrbuf0, xrbuf1)
    sml = (sl0, sl1)
    smr = (sr0, sr1)
    sidx = (sidx0, sidx1)
    msgb = (msgbuf0, msgbuf1)
    sms = (ss0, ss1)

    pltpu.sync_copy(att_hbm.at[pl.ds(2 * c, 2)], attbuf)

    rows = N_TC // NTILE
    pltpu.sync_copy(zro_hbm, acc.at[pl.ds(s * rows, rows)])
    plsc.subcore_barrier()

    att0 = [attbuf[0, pl.ds(16 * k, 16)] for k in range(4)]
    att1 = [attbuf[1, pl.ds(16 * k, 16)] for k in range(4)]
    lane = lax.broadcasted_iota(jnp.int32, (L,), 0)
    m0 = lane == 0
    m1 = lane == 1
    zv = jnp.zeros((L,), jnp.float32)
    coff = c * N_TC
    ngrp = ECH // L

    def fire(g, b):
        off = g * L
        gsrc[b][...] = srcbuf[pl.ds(off, L)] + coff
        gdst[b][...] = dstbuf[pl.ds(off, L)] + coff
        pltpu.async_copy(xl_hbm.at[gsrc[b]], xlb[b], sml[b])
        pltpu.async_copy(xr_hbm.at[gdst[b]], xrb[b], smr[b])

    def wait(b):
        pltpu.make_async_copy(xl_hbm.at[gsrc[b]], xlb[b], sml[b]).wait()
        pltpu.make_async_copy(xr_hbm.at[gdst[b]], xrb[b], smr[b]).wait()

    def wait_s(b):
        pltpu.make_async_copy(msgb[b], acc.at[sidx[b]], sms[b]).wait()

    def compute(g, b):
        wait_s(b)
        sidx[b][...] = dstbuf[pl.ds(g * L, L)]
        for j in range(L):
            xlv = [xlb[b][j, pl.ds(16 * k, 16)] for k in range(8)]
            lrl = []
            for k in range(8):
                z = xlv[k] + xrb[b][j, pl.ds(16 * k, 16)]
                lrl.append(jnp.maximum(z, 0.2 * z))
            s0 = (lrl[0] * att0[0] + lrl[1] * att0[1]
                  + lrl[2] * att0[2] + lrl[3] * att0[3])
            s1 = (lrl[4] * att1[0] + lrl[5] * att1[1]
                  + lrl[6] * att1[2] + lrl[7] * att1[3])
            p0 = jnp.exp(jnp.full((L,), jnp.sum(s0), jnp.float32))
            p1 = jnp.exp(jnp.full((L,), jnp.sum(s1), jnp.float32))
            for k in range(4):
                msgb[b][j, pl.ds(16 * k, 16)] = p0 * xlv[k]
            for k in range(4, 8):
                msgb[b][j, pl.ds(16 * k, 16)] = p1 * xlv[k]
            msgb[b][j, pl.ds(DH, 16)] = jnp.where(m0, p0,
                                                  jnp.where(m1, p1, zv))
        pltpu.async_copy(msgb[b], acc.at[sidx[b]], sms[b], add=True)

    def pair(t, carry):
        fire(2 * t + 1, 1)
        wait(0)
        compute(2 * t, 0)
        fire(jnp.minimum(2 * t + 2, ngrp - 1), 0)
        wait(1)
        compute(2 * t + 1, 1)
        return carry

    for b in range(2):
        for j in range(L):
            for k in range(ROW1 // 16):
                msgb[b][j, pl.ds(16 * k, 16)] = zv
        sidx[b][...] = jnp.zeros((L,), jnp.int32)
        pltpu.async_copy(msgb[b], acc.at[sidx[b]], sms[b], add=True)

    def chunk(ci, carry):
        pltpu.sync_copy(src_hbm.at[pl.ds(base + ci * ECH, ECH)], srcbuf)
        pltpu.sync_copy(dst_hbm.at[pl.ds(base + ci * ECH, ECH)], dstbuf)
        fire(0, 0)
        lax.fori_loop(0, ngrp // 2, pair, 0)
        wait(0)
        return carry

    lax.fori_loop(0, ept // ECH, chunk, 0)
    wait_s(0)
    wait_s(1)
    plsc.subcore_barrier()
    pltpu.sync_copy(acc.at[pl.ds(s * rows, rows)],
                    out_hbm.at[c, pl.ds(s * rows, rows)])


def _sc1(src, dst, xl_cat, xr_cat, att1, zro1):
    fn = functools.partial(
        pl.kernel,
        out_type=jax.ShapeDtypeStruct((2, N_TC, ROW1), jnp.float32),
        mesh=_mesh(),
        compiler_params=pltpu.CompilerParams(needs_layout_passes=False, use_tc_tiling_on_sc=False),
        scratch_types=[
            pltpu.VMEM((ECH,), jnp.int32),
            pltpu.VMEM((ECH,), jnp.int32),
            pltpu.VMEM((2, 64), jnp.float32),
            pltpu.VMEM((L,), jnp.int32),
            pltpu.VMEM((L,), jnp.int32),
            pltpu.VMEM((L,), jnp.int32),
            pltpu.VMEM((L,), jnp.int32),
            pltpu.VMEM((L,), jnp.int32),
            pltpu.VMEM((L,), jnp.int32),
            pltpu.VMEM((L, DH), jnp.float32),
            pltpu.VMEM((L, DH), jnp.float32),
            pltpu.VMEM((L, DH), jnp.float32),
            pltpu.VMEM((L, DH), jnp.float32),
            pltpu.VMEM((L, ROW1), jnp.float32),
            pltpu.VMEM((L, ROW1), jnp.float32),
            pltpu.VMEM_SHARED((N_TC, ROW1), jnp.float32),
            pltpu.SemaphoreType.DMA,
            pltpu.SemaphoreType.DMA,
            pltpu.SemaphoreType.DMA,
            pltpu.SemaphoreType.DMA,
            pltpu.SemaphoreType.DMA,
            pltpu.SemaphoreType.DMA,
        ],
    )(_sc1_body)
    return fn(src, dst, xl_cat, xr_cat, att1, zro1)


def _tc2_body(o1_ref, wl_ref, wr_ref, b1_ref, xl_ref, xr_ref):
    blk = o1_ref[...]
    numA = blk[0, :, 0:DH]
    numB = blk[1, :, 0:DH]
    den = jnp.concatenate(
        [jnp.broadcast_to(blk[0, :, DH:DH + 1], (BN, 64)),
         jnp.broadcast_to(blk[0, :, DH + 1:DH + 2], (BN, 64)),
         jnp.broadcast_to(blk[1, :, DH:DH + 1], (BN, 64)),
         jnp.broadcast_to(blk[1, :, DH + 1:DH + 2], (BN, 64))], axis=1)
    h = jnp.concatenate([numA, numB], axis=1) / (den + 1e-16) + b1_ref[...]
    h1 = jnp.where(h > 0, h, jnp.exp(jnp.minimum(h, 0.0)) - 1.0)
    xl_ref[...] = jnp.dot(h1, wl_ref[...], preferred_element_type=jnp.float32)
    xr_ref[...] = jnp.dot(h1, wr_ref[...], preferred_element_type=jnp.float32)


def _tc2(o1, Wl2, Wr2, b1r):
    return pl.pallas_call(
        _tc2_body,
        grid=(NB,),
        in_specs=[
            pl.BlockSpec((2, BN, ROW1), lambda i: (0, i, 0)),
            pl.BlockSpec((D1, D2), lambda i: (0, 0)),
            pl.BlockSpec((D1, D2), lambda i: (0, 0)),
            pl.BlockSpec((1, D1), lambda i: (0, 0)),
        ],
        out_specs=[
            pl.BlockSpec((BN, D2), lambda i: (i, 0)),
            pl.BlockSpec((BN, D2), lambda i: (i, 0)),
        ],
        out_shape=[
            jax.ShapeDtypeStruct((N_TC, D2), jnp.float32),
            jax.ShapeDtypeStruct((N_TC, D2), jnp.float32),
        ],
    )(o1, Wl2, Wr2, b1r)


def _sc2_body(src_hbm, dst_hbm, xl_hbm, xr_hbm, att_hbm, zro_hbm, out_hbm,
              srcbuf, dstbuf, attbuf, gsrc0, gsrc1, gdst0, gdst1, sidx0, sidx1,
              xlbuf0, xlbuf1, xrbuf0, xrbuf1, msgbuf0, msgbuf1, acc,
              sl0, sl1, sr0, sr1, ss0, ss1):
    c = lax.axis_index("c")
    s = lax.axis_index("s")
    ept = E_TOT // (NSC * NTILE)
    base = (c * NTILE + s) * ept
    gsrc = (gsrc0, gsrc1)
    gdst = (gdst0, gdst1)
    xlb = (xlbuf0, xlbuf1)
    xrb = (xrbuf0, xrbuf1)
    sml = (sl0, sl1)
    smr = (sr0, sr1)
    sidx = (sidx0, sidx1)
    msgb = (msgbuf0, msgbuf1)
    sms = (ss0, ss1)

    pltpu.sync_copy(src_hbm.at[pl.ds(base, ept)], srcbuf)
    pltpu.sync_copy(dst_hbm.at[pl.ds(base, ept)], dstbuf)
    pltpu.sync_copy(att_hbm, attbuf)

    rows = N_TC // NTILE
    pltpu.sync_copy(zro_hbm, acc.at[pl.ds(s * rows, rows)])
    plsc.subcore_barrier()

    att0 = [attbuf[0, pl.ds(16 * k, 16)] for k in range(4)]
    lane = lax.broadcasted_iota(jnp.int32, (L,), 0)
    m0 = lane == 0
    zv = jnp.zeros((L,), jnp.float32)
    ngrp = ept // L

    def fire(g, b):
        off = g * L
        gsrc[b][...] = srcbuf[pl.ds(off, L)]
        gdst[b][...] = dstbuf[pl.ds(off, L)]
        pltpu.async_copy(xl_hbm.at[gsrc[b]], xlb[b], sml[b])
        pltpu.async_copy(xr_hbm.at[gdst[b]], xrb[b], smr[b])

    def wait(b):
        pltpu.make_async_copy(xl_hbm.at[gsrc[b]], xlb[b], sml[b]).wait()
        pltpu.make_async_copy(xr_hbm.at[gdst[b]], xrb[b], smr[b]).wait()

    def wait_s(b):
        pltpu.make_async_copy(msgb[b], acc.at[sidx[b]], sms[b]).wait()

    def compute(g, b):
        wait_s(b)
        sidx[b][...] = dstbuf[pl.ds(g * L, L)]
        for j in range(L):
            xlv = [xlb[b][j, pl.ds(16 * k, 16)] for k in range(4)]
            lrl = []
            for k in range(4):
                z = xlv[k] + xrb[b][j, pl.ds(16 * k, 16)]
                lrl.append(jnp.maximum(z, 0.2 * z))
            s0 = (lrl[0] * att0[0] + lrl[1] * att0[1]
                  + lrl[2] * att0[2] + lrl[3] * att0[3])
            p0 = jnp.exp(jnp.full((L,), jnp.sum(s0), jnp.float32))
            for k in range(4):
                msgb[b][j, pl.ds(16 * k, 16)] = p0 * xlv[k]
            msgb[b][j, pl.ds(D2, 16)] = jnp.where(m0, p0, zv)
        pltpu.async_copy(msgb[b], acc.at[sidx[b]], sms[b], add=True)

    def pair(t, carry):
        fire(2 * t + 1, 1)
        wait(0)
        compute(2 * t, 0)
        fire(jnp.minimum(2 * t + 2, ngrp - 1), 0)
        wait(1)
        compute(2 * t + 1, 1)
        return carry

    for b in range(2):
        for j in range(L):
            for k in range(ROW2 // 16):
                msgb[b][j, pl.ds(16 * k, 16)] = zv
        sidx[b][...] = jnp.zeros((L,), jnp.int32)
        pltpu.async_copy(msgb[b], acc.at[sidx[b]], sms[b], add=True)

    fire(0, 0)
    lax.fori_loop(0, ngrp // 2, pair, 0)
    wait(0)
    compute(ngrp - 1, 0)
    wait_s(0)
    wait_s(1)
    plsc.subcore_barrier()
    pltpu.sync_copy(acc.at[pl.ds(s * rows, rows)],
                    out_hbm.at[c, pl.ds(s * rows, rows)])


def _sc2(src, dst, xl2, xr2, att2, zro2):
    fn = functools.partial(
        pl.kernel,
        out_type=jax.ShapeDtypeStruct((2, N_TC, ROW2), jnp.float32),
        mesh=_mesh(),
        compiler_params=pltpu.CompilerParams(needs_layout_passes=False, use_tc_tiling_on_sc=False),
        scratch_types=[
            pltpu.VMEM((E_TOT // (NSC * NTILE),), jnp.int32),
            pltpu.VMEM((E_TOT // (NSC * NTILE),), jnp.int32),
            pltpu.VMEM((1, 64), jnp.float32),
            pltpu.VMEM((L,), jnp.int32),
            pltpu.VMEM((L,), jnp.int32),
            pltpu.VMEM((L,), jnp.int32),
            pltpu.VMEM((L,), jnp.int32),
            pltpu.VMEM((L,), jnp.int32),
            pltpu.VMEM((L,), jnp.int32),
            pltpu.VMEM((L, D2), jnp.float32),
            pltpu.VMEM((L, D2), jnp.float32),
            pltpu.VMEM((L, D2), jnp.float32),
            pltpu.VMEM((L, D2), jnp.float32),
            pltpu.VMEM((L, ROW2), jnp.float32),
            pltpu.VMEM((L, ROW2), jnp.float32),
            pltpu.VMEM_SHARED((N_TC, ROW2), jnp.float32),
            pltpu.SemaphoreType.DMA,
            pltpu.SemaphoreType.DMA,
            pltpu.SemaphoreType.DMA,
            pltpu.SemaphoreType.DMA,
            pltpu.SemaphoreType.DMA,
            pltpu.SemaphoreType.DMA,
        ],
    )(_sc2_body)
    return fn(src, dst, xl2, xr2, att2, zro2)


def _tc3_body(o2_ref, b2_ref, bt_ref, pegs_ref, move_ref,
              vW1_ref, vb1_ref, vW2_ref, vb2_ref,
              pW1_ref, pb1_ref, pW2_ref, pb2_ref,
              p_ref, v_ref, accbuf):
    i = pl.program_id(0)

    @pl.when(i == 0)
    def _():
        accbuf[...] = jnp.zeros((64, 72), jnp.float32)

    blk = o2_ref[...]
    num = blk[0, :, 0:D2] + blk[1, :, 0:D2]
    den = blk[0, :, D2:D2 + 1] + blk[1, :, D2:D2 + 1]
    h = num / (den + 1e-16) + b2_ref[...]
    h2 = jnp.where(h > 0, h, jnp.exp(jnp.minimum(h, 0.0)) - 1.0)
    bt = bt_ref[0, 0, :]
    onehot = (bt[None, :] == lax.broadcasted_iota(jnp.int32, (64, BN), 0))
    onehot = onehot.astype(jnp.float32)
    ext = jnp.concatenate([h2, jnp.ones((BN, 8), jnp.float32)], axis=1)
    accbuf[...] += jnp.dot(onehot, ext, preferred_element_type=jnp.float32)

    @pl.when(i == NB - 1)
    def _():
        acc = accbuf[...]
        emb = acc[:, 0:64] / jnp.maximum(acc[:, 64:65], 1.0)
        comb = jnp.concatenate(
            [emb, pegs_ref[...], move_ref[...],
             jnp.zeros((64, 6), jnp.float32)], axis=1)
        hv = jnp.maximum(
            jnp.dot(comb, vW1_ref[...], preferred_element_type=jnp.float32)
            + vb1_ref[...], 0.0)
        v_ref[...] = jnp.tanh(
            jnp.dot(hv, vW2_ref[...], preferred_element_type=jnp.float32)
            + vb2_ref[...])
        hp = jnp.maximum(
            jnp.dot(comb, pW1_ref[...], preferred_element_type=jnp.float32)
            + pb1_ref[...], 0.0)
        p_ref[...] = (
            jnp.dot(hp, pW2_ref[...], preferred_element_type=jnp.float32)
            + pb2_ref[...])


def _tc3(o2, b2r, batch_r, pegs, move, vW1p, vb1r, vW2, vb2r, pW1p, pb1r,
         pW2, pb2r):
    A = pW2.shape[1]
    cst = lambda *shape: pl.BlockSpec(shape, lambda i: (0,) * len(shape))
    return pl.pallas_call(
        _tc3_body,
        grid=(NB,),
        in_specs=[
            pl.BlockSpec((2, BN, ROW2), lambda i: (0, i, 0)),
            cst(1, D2),
            pl.BlockSpec((1, 1, BN), lambda i: (i, 0, 0)),
            cst(64, 1), cst(64, 1),
            cst(72, 64), cst(1, 64), cst(64, 1), cst(1, 1),
            cst(72, 64), cst(1, 64), cst(64, A), cst(1, A),
        ],
        out_specs=[
            pl.BlockSpec((64, A), lambda i: (0, 0)),
            pl.BlockSpec((64, 1), lambda i: (0, 0)),
        ],
        out_shape=[
            jax.ShapeDtypeStruct((64, A), jnp.float32),
            jax.ShapeDtypeStruct((64, 1), jnp.float32),
        ],
        scratch_shapes=[pltpu.VMEM((64, 72), jnp.float32)],
    )(o2, b2r, batch_r, pegs, move, vW1p, vb1r, vW2, vb2r, pW1p, pb1r, pW2,
      pb2r)


def kernel(x, edge_index, batch, pegs_left, move_count,
           Wl1, Wr1, att1, b1, Wl2, Wr2, att2, b2,
           vW1, vb1, vW2, vb2, pW1, pb1, pW2, pb2):
    N = x.shape[0]
    B = pegs_left.shape[0]

    x_pad = jnp.pad(x, ((0, N_TC - N), (0, 0)))
    src = edge_index[0]
    dst = edge_index[1]
    batch_r = jnp.pad(batch, (0, N_TC - N),
                      constant_values=B).reshape(NB, 1, BN)
    zro1 = jnp.zeros((N_TC // NTILE, ROW1), jnp.float32)
    zro2 = jnp.zeros((N_TC // NTILE, ROW2), jnp.float32)

    xl_cat, xr_cat = _tc1(x_pad, Wl1, Wr1)
    o1 = _sc1(src, dst, xl_cat, xr_cat, att1, zro1)
    xl2, xr2 = _tc2(o1, Wl2, Wr2, b1.reshape(1, D1))
    o2 = _sc2(src, dst, xl2, xr2, att2, zro2)
    p, v = _tc3(o2, b2.reshape(1, D2), batch_r, pegs_left, move_count,
                jnp.pad(vW1, ((0, 6), (0, 0))), vb1.reshape(1, 64),
                vW2, vb2.reshape(1, 1),
                jnp.pad(pW1, ((0, 6), (0, 0))), pb1.reshape(1, 64),
                pW2, pb2.reshape(1, pW2.shape[1]))
    return (p, v)
